# baseline (speedup 1.0000x reference)
import jax
import jax.numpy as jnp
from jax import lax
from jax.experimental import pallas as pl
from jax.experimental.pallas import tpu as pltpu
from jax.experimental.pallas import tpu_sc as plsc

B, N, D = 128, 576, 768
NC, NS, L = 2, 16, 16
NW = NC * NS
PP = N // NW
CHUNK = PP * D
VECS = CHUNK // L
PB = 2
NBUF = 4
NP = B // PB


def _sc_body(x_hbm, t_hbm, o_hbm, tbl_v,
             b0, b1, b2, b3, si0, si1, si2, si3, so0, so1, so2, so3):
    bufs = (b0, b1, b2, b3)
    sins = (si0, si1, si2, si3)
    souts = (so0, so1, so2, so3)
    wid = lax.axis_index("s") * NC + lax.axis_index("c")
    tbase = wid * CHUNK
    pltpu.sync_copy(t_hbm.at[pl.ds(tbase, CHUNK)], tbl_v)

    def src(p):
        return x_hbm.at[pl.ds(p * PB, PB), pl.ds(tbase, CHUNK)]

    def dst(p):
        return o_hbm.at[pl.ds(p * PB, PB), pl.ds(tbase, CHUNK)]

    def add(buf):
        def add_vec(j, c):
            sl = pl.ds(j * L, L)
            buf[0, sl] = buf[0, sl] + tbl_v[sl]
            buf[1, sl] = buf[1, sl] + tbl_v[sl]
            return c
        lax.fori_loop(0, VECS, add_vec, 0, unroll=4)

    pltpu.async_copy(src(0), bufs[0], sins[0])
    pltpu.async_copy(src(1), bufs[1], sins[1])
    for p in (0, 1):
        pltpu.make_async_copy(src(p), bufs[p], sins[p]).wait()
        pltpu.async_copy(src(p + 2), bufs[p + 2], sins[p + 2])
        add(bufs[p])
        pltpu.async_copy(bufs[p], dst(p), souts[p])

    def group(g, c):
        for k in range(4):
            p = 4 * g + 2 + k
            i = (2 + k) % 4
            j = k % 4
            pltpu.make_async_copy(src(p), bufs[i], sins[i]).wait()
            pltpu.make_async_copy(bufs[j], dst(p - 2), souts[j]).wait()
            pltpu.async_copy(src(p + 2), bufs[j], sins[j])
            add(bufs[i])
            pltpu.async_copy(bufs[i], dst(p), souts[i])
        return c

    lax.fori_loop(0, (NP - 4) // 4, group, 0)

    for p in (NP - 2, NP - 1):
        i = p % 4
        pltpu.make_async_copy(src(p), bufs[i], sins[i]).wait()
        pltpu.make_async_copy(bufs[(p + 2) % 4], dst(p - 2), souts[(p + 2) % 4]).wait()
        add(bufs[i])
        pltpu.async_copy(bufs[i], dst(p), souts[i])
    for p in (NP - 2, NP - 1):
        i = p % 4
        pltpu.make_async_copy(bufs[i], dst(p), souts[i]).wait()


def _sc_call(x_rows, t_flat):
    mesh = plsc.VectorSubcoreMesh(core_axis_name="c", subcore_axis_name="s")
    kfn = pl.kernel(
        _sc_body,
        out_type=jax.ShapeDtypeStruct((B, N * D), jnp.float32),
        mesh=mesh,
        scratch_types=(
            [pltpu.VMEM((CHUNK,), jnp.float32)]
            + [pltpu.VMEM((PB, CHUNK), jnp.float32) for _ in range(NBUF)]
            + [pltpu.SemaphoreType.DMA for _ in range(2 * NBUF)]
        ),
    )
    return kfn(x_rows, t_flat)


def kernel(encoded_patches, position_embedding):
    x_rows = encoded_patches.reshape(B, N * D)
    t_flat = position_embedding.reshape(N * D)
    out = _sc_call(x_rows, t_flat)
    return out.reshape(B, N, D)


# speedup vs baseline: 6.5738x; 6.5738x over previous
import jax
import jax.numpy as jnp
from jax.experimental import pallas as pl
from jax.experimental.pallas import tpu as pltpu


def _add_kernel(x_ref, t_ref, o_ref):
    o_ref[...] = x_ref[...] + t_ref[...][None, :, :]


def kernel(encoded_patches, position_embedding):
    B, N, D = encoded_patches.shape
    BB = 8
    return pl.pallas_call(
        _add_kernel,
        grid=(B // BB,),
        in_specs=[
            pl.BlockSpec((BB, N, D), lambda i: (i, 0, 0)),
            pl.BlockSpec((N, D), lambda i: (0, 0)),
        ],
        out_specs=pl.BlockSpec((BB, N, D), lambda i: (i, 0, 0)),
        out_shape=jax.ShapeDtypeStruct((B, N, D), jnp.float32),
        compiler_params=pltpu.CompilerParams(
            vmem_limit_bytes=120 * 1024 * 1024,
        ),
    )(encoded_patches, position_embedding)
